# baseline (speedup 1.0000x reference)
import functools

import jax
import jax.numpy as jnp
import numpy as np
from jax import lax
from jax.experimental import pallas as pl
from jax.experimental.pallas import tpu as pltpu

TB = 1280
NBLK = 8
NPAD = TB * NBLK
CW = 2 * TB
NCH = NBLK // 2
NA = NBLK * NCH


def _schedule():
    rowL, ch, jA, rowO, em, fB = [], [], [], [], [], []
    for a in range(NBLK):
        cd = a // 2
        order = [c for c in range(NCH) if c != cd] + [cd]
        for j, c in enumerate(order):
            rowL.append(a)
            ch.append(c)
            jA.append(j)
            rowO.append(NBLK - 1)
            em.append(0)
            fB.append(0)
    for a in range(NBLK - 1):
        cmin = (a + 1) // 2
        for c in range(cmin, NCH):
            rowL.append(a)
            ch.append(c)
            jA.append(0)
            rowO.append(a)
            em.append(1 if c == NCH - 1 else 0)
            fB.append(1 if c == cmin else 0)
    return tuple(np.asarray(v, np.int32)
                 for v in (rowL, ch, jA, rowO, em, fB))


HB = TB // 2


def _body(n, rowL_r, ch_r, jA_r, rowO_r, em_r, fB_r,
          L0_ref, L1_ref, xf3_ref, w_ref, b_ref, out_ref,
          y1acc_ref, z_ref, ay_ref, az_ref, acc_ref):
    s = pl.program_id(0)
    in_a = s < NA

    a = rowL_r[s]
    c = ch_r[s]
    j = jA_r[s]
    ao = rowO_r[s]
    emit = em_r[s] == 1
    first = fB_r[s] == 1
    s0 = 2 * c
    s1 = 2 * c + 1

    def _emit_out(y1val, zval):
        w0 = w_ref[0]
        w1 = w_ref[1]
        w2 = w_ref[2]
        out = jnp.dot(xf3_ref[ao], w0 - w2,
                      preferred_element_type=jnp.float32)
        out += jnp.dot(y1val, w1, preferred_element_type=jnp.float32)
        out += jnp.dot(2.0 * zval, w2, preferred_element_type=jnp.float32)
        out_ref[...] = out + b_ref[...]

    def _phaseA_half(h, lo, hi, hw):
        r0, r1 = h * HB, (h + 1) * HB
        ay_ref[r0:r1] += (
            jnp.dot(lo, xf3_ref[s0], preferred_element_type=jnp.float32) +
            jnp.dot(hi, xf3_ref[s1, :hw],
                    preferred_element_type=jnp.float32))

        @pl.when(s0 < a)
        def _lo0():
            az_ref[r0:r1] += jnp.dot(lo, y1acc_ref[s0],
                                     preferred_element_type=jnp.float32)

        @pl.when(s1 < a)
        def _lo1():
            az_ref[r0:r1] += jnp.dot(hi, y1acc_ref[s1, :hw],
                                     preferred_element_type=jnp.float32)

    def _finalizeA(lo0, hi0, lo1, hi1, hw):
        row = lax.broadcasted_iota(jnp.int32, (TB, 1), 0)
        y1_a = ay_ref[...]
        y1_a = jnp.where(
            jnp.logical_or(a < NBLK - 1, row < n - (NBLK - 1) * TB),
            y1_a, 0.0)
        y1acc_ref[a] = y1_a

        @pl.when(a % 2 == 0)
        def _diag_lo():
            z_ref[a, :HB] = az_ref[:HB] + jnp.dot(
                lo0, y1_a, preferred_element_type=jnp.float32)
            z_ref[a, HB:] = az_ref[HB:] + jnp.dot(
                lo1, y1_a, preferred_element_type=jnp.float32)

        @pl.when(a % 2 == 1)
        def _diag_hi():
            z_ref[a, :HB] = az_ref[:HB] + jnp.dot(
                hi0, y1_a[:hw], preferred_element_type=jnp.float32)
            z_ref[a, HB:] = az_ref[HB:] + jnp.dot(
                hi1, y1_a[:hw], preferred_element_type=jnp.float32)

        @pl.when(a == NBLK - 1)
        def _emit_last():
            _emit_out(y1_a, z_ref[a])

    def _phaseB_half(h, lo, hi, hw):
        r0, r1 = h * HB, (h + 1) * HB

        @pl.when(s0 > a)
        def _up0():
            acc_ref[r0:r1] += jnp.dot(lo, y1acc_ref[s0],
                                      preferred_element_type=jnp.float32)

        @pl.when(s1 > a)
        def _up1():
            acc_ref[r0:r1] += jnp.dot(hi, y1acc_ref[s1, :hw],
                                      preferred_element_type=jnp.float32)

    def _dispatch(hw):
        lo0 = L0_ref[:, :TB]
        hi0 = L0_ref[:, TB:TB + hw]
        lo1 = L1_ref[:, :TB]
        hi1 = L1_ref[:, TB:TB + hw]

        @pl.when(in_a)
        def _():
            @pl.when(j == 0)
            def _init():
                ay_ref[...] = jnp.zeros_like(ay_ref)
                az_ref[...] = jnp.zeros_like(az_ref)

            _phaseA_half(0, lo0, hi0, hw)
            _phaseA_half(1, lo1, hi1, hw)

            @pl.when(j == NCH - 1)
            def _fin():
                _finalizeA(lo0, hi0, lo1, hi1, hw)

        @pl.when(jnp.logical_not(in_a))
        def _():
            @pl.when(first)
            def _load():
                acc_ref[...] = z_ref[ao]

            _phaseB_half(0, lo0, hi0, hw)
            _phaseB_half(1, lo1, hi1, hw)

            @pl.when(emit)
            def _do_emit():
                _emit_out(y1acc_ref[ao], acc_ref[...])

    htail = n - (NPAD - CW) - TB

    @pl.when(c < NCH - 1)
    def _interior():
        _dispatch(TB)

    @pl.when(c == NCH - 1)
    def _edge():
        _dispatch(htail)


@jax.jit
def kernel(x, L_cheb, weight, bias):
    tasks, n, c = x.shape
    kdeg = weight.shape[1]
    tc = tasks * c

    xf = jnp.transpose(x, (1, 0, 2)).reshape(n, tc)
    xf3 = jnp.zeros((NPAD, tc), jnp.float32).at[:n].set(xf).reshape(
        NBLK, TB, tc)
    eye = jnp.eye(tasks, dtype=weight.dtype)
    wbd = jnp.einsum('ts,tkio->ksito', eye, weight).reshape(
        kdeg, tasks * c, tasks * weight.shape[-1])
    bias_flat = bias.reshape(1, tasks * bias.shape[-1])

    sched = _schedule()
    nsteps = len(sched[0])

    grid_spec = pltpu.PrefetchScalarGridSpec(
        num_scalar_prefetch=6,
        grid=(nsteps,),
        in_specs=[
            pl.BlockSpec((HB, CW),
                         lambda s, rowL, ch, jA, rowO, em, fB:
                         (2 * rowL[s], ch[s])),
            pl.BlockSpec((HB, CW),
                         lambda s, rowL, ch, jA, rowO, em, fB:
                         (2 * rowL[s] + 1, ch[s])),
            pl.BlockSpec((NBLK, TB, tc), lambda s, *_: (0, 0, 0)),
            pl.BlockSpec(wbd.shape, lambda s, *_: (0, 0, 0)),
            pl.BlockSpec((1, tc), lambda s, *_: (0, 0)),
        ],
        out_specs=pl.BlockSpec((TB, tc),
                               lambda s, rowL, ch, jA, rowO, em, fB:
                               (rowO[s], 0)),
        scratch_shapes=[
            pltpu.VMEM((NBLK, TB, tc), jnp.float32),
            pltpu.VMEM((NBLK, TB, tc), jnp.float32),
            pltpu.VMEM((TB, tc), jnp.float32),
            pltpu.VMEM((TB, tc), jnp.float32),
            pltpu.VMEM((TB, tc), jnp.float32),
        ],
    )

    out_f = pl.pallas_call(
        functools.partial(_body, n),
        grid_spec=grid_spec,
        out_shape=jax.ShapeDtypeStruct((n, tc), jnp.float32),
        compiler_params=pltpu.CompilerParams(
            vmem_limit_bytes=63 * 1024 * 1024),
    )(*sched, L_cheb, L_cheb, xf3, wbd, bias_flat)

    return jnp.transpose(out_f.reshape(n, tasks, c), (1, 0, 2))


# speedup vs baseline: 1.0565x; 1.0565x over previous
import functools

import jax
import jax.numpy as jnp
import numpy as np
from jax import lax
from jax.experimental import pallas as pl
from jax.experimental.pallas import tpu as pltpu

TB = 1280
NBLK = 8
NPAD = TB * NBLK
CW = 2 * TB
NCH = NBLK // 2
NA = NBLK * NCH


def _schedule():
    rowL, ch, jA, rowO, em, fB = [], [], [], [], [], []
    for a in range(NBLK):
        cd = a // 2
        order = [c for c in range(NCH) if c != cd] + [cd]
        for j, c in enumerate(order):
            rowL.append(a)
            ch.append(c)
            jA.append(j)
            rowO.append(NBLK - 1)
            em.append(0)
            fB.append(0)
    for a in range(NBLK - 1):
        cmin = (a + 1) // 2
        for c in range(cmin, NCH):
            rowL.append(a)
            ch.append(c)
            jA.append(0)
            rowO.append(a)
            em.append(1 if c == NCH - 1 else 0)
            fB.append(1 if c == cmin else 0)
    return tuple(np.asarray(v, np.int32)
                 for v in (rowL, ch, jA, rowO, em, fB))


def _body(n, rowL_r, ch_r, jA_r, rowO_r, em_r, fB_r,
          Lc_ref, xf3_ref, w_ref, b_ref, out_ref,
          y1acc_ref, z_ref, ay_ref, az_ref, acc_ref):
    s = pl.program_id(0)
    in_a = s < NA

    a = rowL_r[s]
    c = ch_r[s]
    j = jA_r[s]
    ao = rowO_r[s]
    emit = em_r[s] == 1
    first = fB_r[s] == 1
    s0 = 2 * c
    s1 = 2 * c + 1

    def _emit_out(y1val, zval):
        w0 = w_ref[0]
        w1 = w_ref[1]
        w2 = w_ref[2]
        out = jnp.dot(xf3_ref[ao], w0 - w2,
                      preferred_element_type=jnp.float32)
        out += jnp.dot(y1val, w1, preferred_element_type=jnp.float32)
        out += jnp.dot(2.0 * zval, w2, preferred_element_type=jnp.float32)
        out_ref[...] = out + b_ref[...]

    def _phaseA(lo, hi, hw):
        @pl.when(j == 0)
        def _init():
            ay_ref[...] = jnp.zeros_like(ay_ref)
            az_ref[...] = jnp.zeros_like(az_ref)

        ay_ref[...] += (
            jnp.dot(lo, xf3_ref[s0], preferred_element_type=jnp.float32) +
            jnp.dot(hi, xf3_ref[s1, :hw],
                    preferred_element_type=jnp.float32))

        @pl.when(s0 < a)
        def _lo0():
            az_ref[...] += jnp.dot(lo, y1acc_ref[s0],
                                   preferred_element_type=jnp.float32)

        @pl.when(s1 < a)
        def _lo1():
            az_ref[...] += jnp.dot(hi, y1acc_ref[s1, :hw],
                                   preferred_element_type=jnp.float32)

        @pl.when(j == NCH - 1)
        def _finalize():
            row = lax.broadcasted_iota(jnp.int32, (TB, 1), 0)
            y1_a = ay_ref[...]
            y1_a = jnp.where(
                jnp.logical_or(a < NBLK - 1, row < n - (NBLK - 1) * TB),
                y1_a, 0.0)
            y1acc_ref[a] = y1_a

            @pl.when(a % 2 == 0)
            def _diag_lo():
                z_ref[a] = az_ref[...] + jnp.dot(
                    lo, y1_a, preferred_element_type=jnp.float32)

            @pl.when(a % 2 == 1)
            def _diag_hi():
                z_ref[a] = az_ref[...] + jnp.dot(
                    hi, y1_a[:hw], preferred_element_type=jnp.float32)

            @pl.when(a == NBLK - 1)
            def _emit_last():
                _emit_out(y1_a, z_ref[a])

    def _phaseB(lo, hi, hw):
        @pl.when(first)
        def _load():
            acc_ref[...] = z_ref[ao]

        @pl.when(s0 > a)
        def _up0():
            acc_ref[...] += jnp.dot(lo, y1acc_ref[s0],
                                    preferred_element_type=jnp.float32)

        @pl.when(s1 > a)
        def _up1():
            acc_ref[...] += jnp.dot(hi, y1acc_ref[s1, :hw],
                                    preferred_element_type=jnp.float32)

        @pl.when(emit)
        def _do_emit():
            _emit_out(y1acc_ref[ao], acc_ref[...])

    def _dispatch(lo, hi, hw):
        @pl.when(in_a)
        def _():
            _phaseA(lo, hi, hw)

        @pl.when(jnp.logical_not(in_a))
        def _():
            _phaseB(lo, hi, hw)

    htail = n - (NPAD - CW) - TB

    @pl.when(c < NCH - 1)
    def _interior():
        _dispatch(Lc_ref[:, :TB], Lc_ref[:, TB:], TB)

    @pl.when(c == NCH - 1)
    def _edge():
        _dispatch(Lc_ref[:, :TB], Lc_ref[:, TB:TB + htail], htail)


@jax.jit
def kernel(x, L_cheb, weight, bias):
    tasks, n, c = x.shape
    kdeg = weight.shape[1]
    tc = tasks * c

    xf = jnp.transpose(x, (1, 0, 2)).reshape(n, tc)
    xf3 = jnp.zeros((NPAD, tc), jnp.float32).at[:n].set(xf).reshape(
        NBLK, TB, tc)
    eye = jnp.eye(tasks, dtype=weight.dtype)
    wbd = jnp.einsum('ts,tkio->ksito', eye, weight).reshape(
        kdeg, tasks * c, tasks * weight.shape[-1])
    bias_flat = bias.reshape(1, tasks * bias.shape[-1])

    sched = _schedule()
    nsteps = len(sched[0])

    grid_spec = pltpu.PrefetchScalarGridSpec(
        num_scalar_prefetch=6,
        grid=(nsteps,),
        in_specs=[
            pl.BlockSpec((TB, CW),
                         lambda s, rowL, ch, jA, rowO, em, fB:
                         (rowL[s], ch[s])),
            pl.BlockSpec((NBLK, TB, tc), lambda s, *_: (0, 0, 0)),
            pl.BlockSpec(wbd.shape, lambda s, *_: (0, 0, 0)),
            pl.BlockSpec((1, tc), lambda s, *_: (0, 0)),
        ],
        out_specs=pl.BlockSpec((TB, tc),
                               lambda s, rowL, ch, jA, rowO, em, fB:
                               (rowO[s], 0)),
        scratch_shapes=[
            pltpu.VMEM((NBLK, TB, tc), jnp.float32),
            pltpu.VMEM((NBLK, TB, tc), jnp.float32),
            pltpu.VMEM((TB, tc), jnp.float32),
            pltpu.VMEM((TB, tc), jnp.float32),
            pltpu.VMEM((TB, tc), jnp.float32),
        ],
    )

    out_f = pl.pallas_call(
        functools.partial(_body, n),
        grid_spec=grid_spec,
        out_shape=jax.ShapeDtypeStruct((n, tc), jnp.float32),
        compiler_params=pltpu.CompilerParams(
            vmem_limit_bytes=63 * 1024 * 1024),
    )(*sched, L_cheb, xf3, wbd, bias_flat)

    return jnp.transpose(out_f.reshape(n, tasks, c), (1, 0, 2))


# speedup vs baseline: 1.0696x; 1.0124x over previous
import functools

import jax
import jax.numpy as jnp
from jax import lax
from jax.experimental import pallas as pl
from jax.experimental.pallas import tpu as pltpu

TB = 1280
NBLK = 8
NPAD = TB * NBLK
CW = 2 * TB
NCH = NBLK // 2


def _chunkA_of(a, j):
    cd = a // 2
    last = j == NCH - 1
    c = j + (j >= cd).astype(jnp.int32)
    return jnp.where(last, cd, c)


def _passA_body(n, L_ref, xf_ref, y1_ref, z_ref, y1acc_ref, ay_ref, az_ref):
    a = pl.program_id(0)
    j = pl.program_id(1)
    c = _chunkA_of(a, j)
    edge = c == NCH - 1
    ntail = n - (NCH - 1) * CW

    @pl.when(j == 0)
    def _init():
        ay_ref[...] = jnp.zeros_like(ay_ref)
        az_ref[...] = jnp.zeros_like(az_ref)

    def _work(Lc):
        ay_ref[...] += jnp.dot(Lc, xf_ref[c],
                               preferred_element_type=jnp.float32)
        s0 = 2 * c
        s1 = 2 * c + 1

        @pl.when(s0 < a)
        def _lo0():
            az_ref[...] += jnp.dot(Lc[:, :TB], y1acc_ref[s0],
                                   preferred_element_type=jnp.float32)

        @pl.when(s1 < a)
        def _lo1():
            az_ref[...] += jnp.dot(Lc[:, TB:], y1acc_ref[s1],
                                   preferred_element_type=jnp.float32)

        @pl.when(j == NCH - 1)
        def _finalize():
            row = lax.broadcasted_iota(jnp.int32, (TB, 1), 0)
            y1_a = ay_ref[...]
            y1_a = jnp.where(
                jnp.logical_or(a < NBLK - 1, row < n - (NBLK - 1) * TB),
                y1_a, 0.0)
            y1acc_ref[a] = y1_a
            y1_ref[0] = y1_a

            @pl.when(a % 2 == 0)
            def _diag_lo():
                z_ref[0] = az_ref[...] + jnp.dot(
                    Lc[:, :TB], y1_a, preferred_element_type=jnp.float32)

            @pl.when(a % 2 == 1)
            def _diag_hi():
                z_ref[0] = az_ref[...] + jnp.dot(
                    Lc[:, TB:], y1_a, preferred_element_type=jnp.float32)

    @pl.when(jnp.logical_not(edge))
    def _body():
        _work(L_ref[...])

    @pl.when(edge)
    def _body_edge():
        col = lax.broadcasted_iota(jnp.int32, (TB, CW), 1)
        _work(jnp.where(col < ntail, L_ref[...], 0.0))


def _passB_body(n, L_ref, y1_ref, z_ref, xf_ref, w_ref, b_ref, out_ref,
                acc_ref):
    a = pl.program_id(0)
    j = pl.program_id(1)
    jmin = a + 1
    edge = j == NBLK - 1
    ntail = n - (NBLK - 1) * TB

    @pl.when(j == 0)
    def _load():
        acc_ref[...] = z_ref[0]

    @pl.when(jnp.logical_and(j >= jmin, jnp.logical_not(edge)))
    def _upper():
        acc_ref[...] += jnp.dot(L_ref[...], y1_ref[j],
                                preferred_element_type=jnp.float32)

    @pl.when(edge)
    def _edge_and_emit():
        @pl.when(j >= jmin)
        def _upper_edge():
            col = lax.broadcasted_iota(jnp.int32, (TB, TB), 1)
            Lc = jnp.where(col < ntail, L_ref[...], 0.0)
            acc_ref[...] += jnp.dot(Lc, y1_ref[j],
                                    preferred_element_type=jnp.float32)

        w0 = w_ref[0]
        w1 = w_ref[1]
        w2 = w_ref[2]
        out = jnp.dot(xf_ref[a], w0 - w2, preferred_element_type=jnp.float32)
        out += jnp.dot(y1_ref[a], w1, preferred_element_type=jnp.float32)
        out += jnp.dot(2.0 * acc_ref[...], w2,
                       preferred_element_type=jnp.float32)
        out_ref[...] = out + b_ref[...]


@jax.jit
def kernel(x, L_cheb, weight, bias):
    tasks, n, c = x.shape
    kdeg = weight.shape[1]
    tc = tasks * c

    xf = jnp.transpose(x, (1, 0, 2)).reshape(n, tc)
    xfp = jnp.zeros((NPAD, tc), jnp.float32).at[:n].set(xf)
    xf3 = xfp.reshape(NBLK, TB, tc)
    xfc = xfp.reshape(NCH, CW, tc)
    eye = jnp.eye(tasks, dtype=weight.dtype)
    wbd = jnp.einsum('ts,tkio->ksito', eye, weight).reshape(
        kdeg, tasks * c, tasks * weight.shape[-1])
    bias_flat = bias.reshape(1, tasks * bias.shape[-1])

    y13, z3 = pl.pallas_call(
        functools.partial(_passA_body, n),
        grid=(NBLK, NCH),
        in_specs=[
            pl.BlockSpec((TB, CW), lambda a, j: (a, _chunkA_of(a, j))),
            pl.BlockSpec((NCH, CW, tc), lambda a, j: (0, 0, 0)),
        ],
        out_specs=[
            pl.BlockSpec((1, TB, tc), lambda a, j: (a, 0, 0)),
            pl.BlockSpec((1, TB, tc), lambda a, j: (a, 0, 0)),
        ],
        out_shape=[
            jax.ShapeDtypeStruct((NBLK, TB, tc), jnp.float32),
            jax.ShapeDtypeStruct((NBLK, TB, tc), jnp.float32),
        ],
        scratch_shapes=[
            pltpu.VMEM((NBLK, TB, tc), jnp.float32),
            pltpu.VMEM((TB, tc), jnp.float32),
            pltpu.VMEM((TB, tc), jnp.float32),
        ],
    )(L_cheb, xfc)

    out_f = pl.pallas_call(
        functools.partial(_passB_body, n),
        grid=(NBLK, NBLK),
        in_specs=[
            pl.BlockSpec(
                (TB, TB),
                lambda a, j: (a, jnp.minimum(jnp.maximum(j, a + 1),
                                             NBLK - 1))),
            pl.BlockSpec((NBLK, TB, tc), lambda a, j: (0, 0, 0)),
            pl.BlockSpec((1, TB, tc), lambda a, j: (a, 0, 0)),
            pl.BlockSpec((NBLK, TB, tc), lambda a, j: (0, 0, 0)),
            pl.BlockSpec(wbd.shape, lambda a, j: (0, 0, 0)),
            pl.BlockSpec((1, tc), lambda a, j: (0, 0)),
        ],
        out_specs=pl.BlockSpec((TB, tc), lambda a, j: (a, 0)),
        out_shape=jax.ShapeDtypeStruct((n, tc), jnp.float32),
        scratch_shapes=[pltpu.VMEM((TB, tc), jnp.float32)],
    )(L_cheb, y13, z3, xf3, wbd, bias_flat)

    return jnp.transpose(out_f.reshape(n, tasks, c), (1, 0, 2))
